# Initial kernel scaffold; baseline (speedup 1.0000x reference)
import jax
import jax.numpy as jnp
from jax.experimental import pallas as pl


def kernel(user_idx, item_idx, user_hist, embed_global, embed_target, embed_hist, aff_user_gmf, aff_item_gmf, aff_user_mlp, aff_item_mlp, W1, b1, W2, b2, W3, b3, W_logit, b_logit):
    raise NotImplementedError("write your pallas kernel here")


# speedup vs baseline: 8.1025x; 8.1025x over previous
import functools
import math

import jax
import jax.numpy as jnp
from jax import lax
from jax.experimental import pallas as pl
from jax.experimental.pallas import tpu as pltpu
from jax.experimental.pallas import tpu_sc as plsc

NC, NS = 2, 16
NW = NC * NS
L = 128


def _sc_mesh():
    return plsc.VectorSubcoreMesh(
        core_axis_name="c", subcore_axis_name="s",
        num_cores=NC, num_subcores=NS)


def _wid():
    return lax.axis_index("s") * NC + lax.axis_index("c")


def _sc_gather_user(uidx2, iidx2, hist_t, ucat_t, et_t):
    nrow, _ = uidx2.shape
    B = nrow * L
    nb = B // NW
    nch = nb // L

    @functools.partial(
        pl.kernel,
        out_type=(jax.ShapeDtypeStruct((B, L), jnp.int32),
                  jax.ShapeDtypeStruct((B, L), jnp.float32),
                  jax.ShapeDtypeStruct((B, L), jnp.float32)),
        mesh=_sc_mesh(),
        scratch_types=[pltpu.VMEM((nch, L), jnp.int32),
                       pltpu.VMEM((nch, L), jnp.int32),
                       pltpu.VMEM((L, L), jnp.int32),
                       pltpu.VMEM((L, L), jnp.float32),
                       pltpu.VMEM((L, L), jnp.float32),
                       pltpu.SemaphoreType.DMA],
    )
    def k(uidx_h, iidx_h, hist_h, ucat_h, et_h,
          hist_o, uc_o, et_o, uv, iv, hv, ucv, etv, sem):
        w = _wid()
        pltpu.sync_copy(uidx_h.at[pl.ds(w * nch, nch)], uv)
        pltpu.sync_copy(iidx_h.at[pl.ds(w * nch, nch)], iv)
        for c in range(nch):
            base = w * nb + c * L
            cps = [pltpu.async_copy(hist_h.at[uv.at[c]], hv, sem),
                   pltpu.async_copy(ucat_h.at[uv.at[c]], ucv, sem),
                   pltpu.async_copy(et_h.at[iv.at[c]], etv, sem)]
            for cp in cps:
                cp.wait()
            pltpu.sync_copy(hv, hist_o.at[pl.ds(base, L)])
            pltpu.sync_copy(ucv, uc_o.at[pl.ds(base, L)])
            pltpu.sync_copy(etv, et_o.at[pl.ds(base, L)])

    return k(uidx2, iidx2, hist_t, ucat_t, et_t)


def _sc_gather_items(ridx2, acat_t, eh_t):
    nrow, _ = ridx2.shape
    T = nrow * L
    nb = T // NW
    nch = nb // L
    G = 4

    @functools.partial(
        pl.kernel,
        out_type=(jax.ShapeDtypeStruct((T, L), jnp.float32),
                  jax.ShapeDtypeStruct((T, L), jnp.float32)),
        mesh=_sc_mesh(),
        scratch_types=[pltpu.VMEM((nch, L), jnp.int32),
                       pltpu.VMEM((G * L, L), jnp.float32),
                       pltpu.SemaphoreType.DMA],
    )
    def k(ridx_h, acat_h, eh_h, ac_o, eh_o, idxv, buf, sem):
        w = _wid()
        pltpu.sync_copy(ridx_h.at[pl.ds(w * nch, nch)], idxv)
        wbase = w * nb

        def gpass(tab_h, out_o):
            def grp(g, carry):
                cps = []
                for t in range(G):
                    cps.append(pltpu.async_copy(
                        tab_h.at[idxv.at[g * G + t]],
                        buf.at[pl.ds(t * L, L)], sem))
                for cp in cps:
                    cp.wait()
                pltpu.sync_copy(buf, out_o.at[pl.ds(wbase + g * (G * L),
                                                    G * L)])
                return carry
            lax.fori_loop(0, nch // G, grp, 0)

        gpass(acat_h, ac_o)
        gpass(eh_h, eh_o)

    return k(ridx2, acat_t, eh_t)


def _tc_maxlen_body(r_ref, out_ref, *, H, PAD):
    r = r_ref[:, :H]
    lens = jnp.sum((r != PAD).astype(jnp.int32), axis=1)
    bm = jnp.max(lens)

    @pl.when(pl.program_id(0) == 0)
    def _():
        out_ref[0, 0] = 1

    out_ref[0, 0] = jnp.maximum(out_ref[0, 0], bm)


def _tc_maxlen(histB, H, PAD):
    B, HP = histB.shape
    BB = 2048
    return pl.pallas_call(
        functools.partial(_tc_maxlen_body, H=H, PAD=PAD),
        grid=(B // BB,),
        in_specs=[pl.BlockSpec((BB, HP), lambda i: (i, 0))],
        out_specs=pl.BlockSpec(memory_space=pltpu.SMEM),
        out_shape=jax.ShapeDtypeStruct((1, 1), jnp.int32),
        compiler_params=pltpu.CompilerParams(
            dimension_semantics=("arbitrary",)),
    )(histB)


def _tc_main_body(r_ref, item_ref, ucat_ref, et_ref, acat_ref, eh_ref,
                  W1_ref, b1_ref, W2_ref, b2_ref, W3_ref, b3_ref,
                  g_ref, wl_ref, bl_ref, ml_ref, out_ref, *, H, PAD, BB):
    F = et_ref.shape[1]
    half = F // 2
    T = BB * H
    r = r_ref[:, :H]
    umlp = ucat_ref[:, half:]
    ugmf = ucat_ref[:, :half]
    am = acat_ref[:, half:]
    zu = jnp.dot(umlp, W1_ref[half:, :],
                 preferred_element_type=jnp.float32) + b1_ref[...]
    x1 = jnp.dot(am, W1_ref[:half, :],
                 preferred_element_type=jnp.float32)
    h1 = jnp.maximum(x1.reshape(BB, H, x1.shape[1]) + zu[:, None, :], 0.0)
    h1f = h1.reshape(T, x1.shape[1])
    h2 = jnp.maximum(jnp.dot(h1f, W2_ref[...],
                             preferred_element_type=jnp.float32)
                     + b2_ref[...], 0.0)
    h3 = jnp.maximum(jnp.dot(h2, W3_ref[...],
                             preferred_element_type=jnp.float32)
                     + b3_ref[...], 0.0)
    g1 = g_ref[0, :half]
    g2 = g_ref[0, half:]
    s_mlp = jnp.sum(h3.reshape(BB, H, half) * g2[None, None, :], axis=2)
    gmf3 = ugmf[:, None, :] * acat_ref[:, :half].reshape(BB, H, half)
    s_gmf = jnp.sum(gmf3 * g1[None, None, :], axis=2)
    scores = (s_gmf + s_mlp) * (1.0 / math.sqrt(F))
    item = item_ref[...]
    bad = (r == item) | (r == PAD)
    scores = jnp.where(bad, jnp.float32(-1e9), scores)
    col = lax.broadcasted_iota(jnp.int32, (BB, H), 1)
    scores = jnp.where(col >= ml_ref[0, 0], jnp.float32(-2e9), scores)
    m = jnp.max(scores, axis=1, keepdims=True)
    e = jnp.exp(scores - m)
    w = e / jnp.sum(e, axis=1, keepdims=True)
    q = et_ref[...] * wl_ref[...]
    vdot = jnp.sum(eh_ref[...].reshape(BB, H, F) * q[:, None, :], axis=2)
    out_ref[...] = jnp.sum(w * vdot, axis=1, keepdims=True) + bl_ref[...]


def _tc_main(histB, item2d, ucat, et, acat, eh,
             W1r, b1, W2, b2, W3, b3, g2d, wl2d, bl2d, maxlen, H, PAD):
    B, HP = histB.shape
    F = et.shape[1]
    D1 = W1r.shape[1]
    BB = 128
    TB = BB * H
    grid = (B // BB,)
    full = lambda s: pl.BlockSpec(s, lambda i: tuple(0 for _ in s))
    return pl.pallas_call(
        functools.partial(_tc_main_body, H=H, PAD=PAD, BB=BB),
        grid=grid,
        in_specs=[
            pl.BlockSpec((BB, HP), lambda i: (i, 0)),
            pl.BlockSpec((BB, 1), lambda i: (i, 0)),
            pl.BlockSpec((BB, F), lambda i: (i, 0)),
            pl.BlockSpec((BB, F), lambda i: (i, 0)),
            pl.BlockSpec((TB, F), lambda i: (i, 0)),
            pl.BlockSpec((TB, F), lambda i: (i, 0)),
            full((F, D1)),
            full((1, D1)),
            full(W2.shape),
            full((1, W2.shape[1])),
            full(W3.shape),
            full((1, W3.shape[1])),
            full((1, F)),
            full((1, F)),
            full((1, 1)),
            pl.BlockSpec(memory_space=pltpu.SMEM),
        ],
        out_specs=pl.BlockSpec((BB, 1), lambda i: (i, 0)),
        out_shape=jax.ShapeDtypeStruct((B, 1), jnp.float32),
        compiler_params=pltpu.CompilerParams(
            dimension_semantics=("arbitrary",)),
    )(histB, item2d, ucat, et, acat, eh,
      W1r, b1.reshape(1, -1), W2, b2.reshape(1, -1), W3, b3.reshape(1, -1),
      g2d, wl2d, bl2d, maxlen)


def kernel(user_idx, item_idx, user_hist, embed_global, embed_target,
           embed_hist, aff_user_gmf, aff_item_gmf, aff_user_mlp,
           aff_item_mlp, W1, b1, W2, b2, W3, b3, W_logit, b_logit):
    B = user_idx.shape[0]
    H = user_hist.shape[1]
    F = embed_global.shape[0]
    PAD = embed_target.shape[0] - 1
    half = F // 2

    user_idx = user_idx.astype(jnp.int32)
    item_idx = item_idx.astype(jnp.int32)
    hist_t = jnp.pad(user_hist.astype(jnp.int32), ((0, 0), (0, L - H)),
                     constant_values=PAD)
    ucat_t = jnp.concatenate([aff_user_gmf, aff_user_mlp], axis=1)
    acat_t = jnp.concatenate([aff_item_gmf, aff_item_mlp], axis=1)
    W1r = jnp.concatenate([W1[half:, :], W1[:half, :]], axis=0)

    uidx2 = user_idx.reshape(-1, 128)
    iidx2 = item_idx.reshape(-1, 128)

    histB, ucat, et = _sc_gather_user(uidx2, iidx2, hist_t, ucat_t,
                                      embed_target)

    ridx2 = histB[:, :H].reshape(-1, 128)
    acat, eh = _sc_gather_items(ridx2, acat_t, embed_hist)

    maxlen = _tc_maxlen(histB, H, PAD)

    logit2 = _tc_main(histB, item_idx.reshape(-1, 1), ucat, et, acat, eh,
                      W1r, b1, W2, b2, W3, b3,
                      embed_global.reshape(1, -1),
                      W_logit.reshape(1, -1), b_logit.reshape(1, 1),
                      maxlen, H, PAD)
    return logit2[:, 0]


# speedup vs baseline: 12.2706x; 1.5144x over previous
import functools
import math

import jax
import jax.numpy as jnp
from jax import lax
from jax.experimental import pallas as pl
from jax.experimental.pallas import tpu as pltpu
from jax.experimental.pallas import tpu_sc as plsc

NC, NS = 2, 16
NW = NC * NS
L = 128


def _sc_mesh():
    return plsc.VectorSubcoreMesh(
        core_axis_name="c", subcore_axis_name="s",
        num_cores=NC, num_subcores=NS)


def _wid():
    return lax.axis_index("s") * NC + lax.axis_index("c")


def _sc_gather_user(uidx2, iidx2, hist_t, ucat_t, et_t):
    nrow, _ = uidx2.shape
    B = nrow * L
    nb = B // NW
    nch = nb // L

    @functools.partial(
        pl.kernel,
        out_type=(jax.ShapeDtypeStruct((B, L), jnp.int32),
                  jax.ShapeDtypeStruct((B, L), jnp.float32),
                  jax.ShapeDtypeStruct((B, L), jnp.float32)),
        mesh=_sc_mesh(),
        scratch_types=[pltpu.VMEM((nch, L), jnp.int32),
                       pltpu.VMEM((nch, L), jnp.int32),
                       pltpu.VMEM((L, L), jnp.int32),
                       pltpu.VMEM((L, L), jnp.float32),
                       pltpu.VMEM((L, L), jnp.float32),
                       pltpu.SemaphoreType.DMA],
    )
    def k(uidx_h, iidx_h, hist_h, ucat_h, et_h,
          hist_o, uc_o, et_o, uv, iv, hv, ucv, etv, sem):
        w = _wid()
        pltpu.sync_copy(uidx_h.at[pl.ds(w * nch, nch)], uv)
        pltpu.sync_copy(iidx_h.at[pl.ds(w * nch, nch)], iv)
        for c in range(nch):
            base = w * nb + c * L
            cps = [pltpu.async_copy(hist_h.at[uv.at[c]], hv, sem),
                   pltpu.async_copy(ucat_h.at[uv.at[c]], ucv, sem),
                   pltpu.async_copy(et_h.at[iv.at[c]], etv, sem)]
            for cp in cps:
                cp.wait()
            pltpu.sync_copy(hv, hist_o.at[pl.ds(base, L)])
            pltpu.sync_copy(ucv, uc_o.at[pl.ds(base, L)])
            pltpu.sync_copy(etv, et_o.at[pl.ds(base, L)])

    return k(uidx2, iidx2, hist_t, ucat_t, et_t)


def _sc_gather_items(ridx2, acat_t, eh_t):
    nrow, _ = ridx2.shape
    T = nrow * L
    nb = T // NW
    nch = nb // L
    G = 4

    @functools.partial(
        pl.kernel,
        out_type=(jax.ShapeDtypeStruct((T, L), jnp.float32),
                  jax.ShapeDtypeStruct((T, L), jnp.float32)),
        mesh=_sc_mesh(),
        scratch_types=[pltpu.VMEM((nch, L), jnp.int32),
                       pltpu.VMEM((G * L, L), jnp.float32),
                       pltpu.SemaphoreType.DMA],
    )
    def k(ridx_h, acat_h, eh_h, ac_o, eh_o, idxv, buf, sem):
        w = _wid()
        pltpu.sync_copy(ridx_h.at[pl.ds(w * nch, nch)], idxv)
        wbase = w * nb

        def gpass(tab_h, out_o):
            def grp(g, carry):
                cps = []
                for t in range(G):
                    cps.append(pltpu.async_copy(
                        tab_h.at[idxv.at[g * G + t]],
                        buf.at[pl.ds(t * L, L)], sem))
                for cp in cps:
                    cp.wait()
                pltpu.sync_copy(buf, out_o.at[pl.ds(wbase + g * (G * L),
                                                    G * L)])
                return carry
            lax.fori_loop(0, nch // G, grp, 0)

        gpass(acat_h, ac_o)
        gpass(eh_h, eh_o)

    return k(ridx2, acat_t, eh_t)


def _tc_maxlen_body(rT_ref, out_ref, *, PAD):
    r = rT_ref[...]
    lens = jnp.sum((r != PAD).astype(jnp.int32), axis=0)
    bm = jnp.max(lens)

    @pl.when(pl.program_id(0) == 0)
    def _():
        out_ref[0, 0] = 1

    out_ref[0, 0] = jnp.maximum(out_ref[0, 0], bm)


def _tc_maxlen(histT, PAD):
    H, B = histT.shape
    BB = 2048
    return pl.pallas_call(
        functools.partial(_tc_maxlen_body, PAD=PAD),
        grid=(B // BB,),
        in_specs=[pl.BlockSpec((H, BB), lambda i: (0, i))],
        out_specs=pl.BlockSpec(memory_space=pltpu.SMEM),
        out_shape=jax.ShapeDtypeStruct((1, 1), jnp.int32),
        compiler_params=pltpu.CompilerParams(
            dimension_semantics=("arbitrary",)),
    )(histT)


def _tc_main_body(rT_ref, item_ref, ucat_ref, et_ref, acat_ref, eh_ref,
                  W1_ref, b1_ref, W2_ref, b2_ref, W3_ref, b3_ref,
                  g_ref, wl_ref, bl_ref, ml_ref, out_ref, *, H, PAD, BB):
    F = et_ref.shape[1]
    half = F // 2
    T = BB * H
    rT = rT_ref[...]
    ucat = ucat_ref[...]
    umlp = ucat[:, half:]
    ugmf = ucat[:, :half]
    acat3 = acat_ref[...]
    am = acat3[:, :, half:].reshape(T, half)
    zu = jnp.dot(umlp, W1_ref[half:, :],
                 preferred_element_type=jnp.float32) + b1_ref[...]
    x1 = jnp.dot(am, W1_ref[:half, :],
                 preferred_element_type=jnp.float32)
    h1 = jnp.maximum(x1.reshape(H, BB, x1.shape[1]) + zu[None, :, :], 0.0)
    h1f = h1.reshape(T, x1.shape[1])
    h2 = jnp.maximum(jnp.dot(h1f, W2_ref[...],
                             preferred_element_type=jnp.float32)
                     + b2_ref[...], 0.0)
    h3 = jnp.maximum(jnp.dot(h2, W3_ref[...],
                             preferred_element_type=jnp.float32)
                     + b3_ref[...], 0.0)
    g1 = g_ref[0, :half]
    g2 = g_ref[0, half:]
    s_mlp = jnp.sum(h3.reshape(H, BB, half) * g2[None, None, :], axis=2)
    ug1 = ugmf * g1[None, :]
    s_gmf = jnp.sum(ug1[None, :, :] * acat3[:, :, :half], axis=2)
    scores = (s_gmf + s_mlp) * (1.0 / math.sqrt(F))
    item = item_ref[...]
    bad = (rT == item) | (rT == PAD)
    scores = jnp.where(bad, jnp.float32(-1e9), scores)
    row = lax.broadcasted_iota(jnp.int32, (H, BB), 0)
    scores = jnp.where(row >= ml_ref[0, 0], jnp.float32(-2e9), scores)
    m = jnp.max(scores, axis=0, keepdims=True)
    e = jnp.exp(scores - m)
    w = e / jnp.sum(e, axis=0, keepdims=True)
    q = et_ref[...] * wl_ref[...]
    vdot = jnp.sum(eh_ref[...] * q[None, :, :], axis=2)
    out_ref[...] = jnp.sum(w * vdot, axis=0, keepdims=True) + bl_ref[...]


def _tc_main(histT, item_row, ucat, et, acat3, eh3,
             W1r, b1, W2, b2, W3, b3, g2d, wl2d, bl2d, maxlen, H, PAD):
    _, B = histT.shape
    F = et.shape[1]
    D1 = W1r.shape[1]
    BB = 128
    grid = (B // BB,)
    full = lambda s: pl.BlockSpec(s, lambda i: tuple(0 for _ in s))
    return pl.pallas_call(
        functools.partial(_tc_main_body, H=H, PAD=PAD, BB=BB),
        grid=grid,
        in_specs=[
            pl.BlockSpec((H, BB), lambda i: (0, i)),
            pl.BlockSpec((1, BB), lambda i: (0, i)),
            pl.BlockSpec((BB, F), lambda i: (i, 0)),
            pl.BlockSpec((BB, F), lambda i: (i, 0)),
            pl.BlockSpec((H, BB, F), lambda i: (0, i, 0)),
            pl.BlockSpec((H, BB, F), lambda i: (0, i, 0)),
            full((F, D1)),
            full((1, D1)),
            full(W2.shape),
            full((1, W2.shape[1])),
            full(W3.shape),
            full((1, W3.shape[1])),
            full((1, F)),
            full((1, F)),
            full((1, 1)),
            pl.BlockSpec(memory_space=pltpu.SMEM),
        ],
        out_specs=pl.BlockSpec((1, BB), lambda i: (0, i)),
        out_shape=jax.ShapeDtypeStruct((1, B), jnp.float32),
        compiler_params=pltpu.CompilerParams(
            dimension_semantics=("arbitrary",)),
    )(histT, item_row, ucat, et, acat3, eh3,
      W1r, b1.reshape(1, -1), W2, b2.reshape(1, -1), W3, b3.reshape(1, -1),
      g2d, wl2d, bl2d, maxlen)


def kernel(user_idx, item_idx, user_hist, embed_global, embed_target,
           embed_hist, aff_user_gmf, aff_item_gmf, aff_user_mlp,
           aff_item_mlp, W1, b1, W2, b2, W3, b3, W_logit, b_logit):
    B = user_idx.shape[0]
    H = user_hist.shape[1]
    F = embed_global.shape[0]
    PAD = embed_target.shape[0] - 1
    half = F // 2

    user_idx = user_idx.astype(jnp.int32)
    item_idx = item_idx.astype(jnp.int32)
    hist_t = jnp.pad(user_hist.astype(jnp.int32), ((0, 0), (0, L - H)),
                     constant_values=PAD)
    ucat_t = jnp.concatenate([aff_user_gmf, aff_user_mlp], axis=1)
    acat_t = jnp.concatenate([aff_item_gmf, aff_item_mlp], axis=1)
    W1r = jnp.concatenate([W1[half:, :], W1[:half, :]], axis=0)

    uidx2 = user_idx.reshape(-1, 128)
    iidx2 = item_idx.reshape(-1, 128)

    histB, ucat, et = _sc_gather_user(uidx2, iidx2, hist_t, ucat_t,
                                      embed_target)

    histT = histB[:, :H].T
    ridx2 = histT.reshape(-1, 128)
    acat, eh = _sc_gather_items(ridx2, acat_t, embed_hist)
    acat3 = acat.reshape(H, B, F)
    eh3 = eh.reshape(H, B, F)

    maxlen = _tc_maxlen(histT, PAD)

    logit2 = _tc_main(histT, item_idx.reshape(1, -1), ucat, et, acat3, eh3,
                      W1r, b1, W2, b2, W3, b3,
                      embed_global.reshape(1, -1),
                      W_logit.reshape(1, -1), b_logit.reshape(1, 1),
                      maxlen, H, PAD)
    return logit2[0]


# speedup vs baseline: 18.4336x; 1.5023x over previous
import functools
import math

import jax
import jax.numpy as jnp
from jax import lax
from jax.experimental import pallas as pl
from jax.experimental.pallas import tpu as pltpu
from jax.experimental.pallas import tpu_sc as plsc

NC, NS = 2, 16
NW = NC * NS
L = 128


def _sc_mesh():
    return plsc.VectorSubcoreMesh(
        core_axis_name="c", subcore_axis_name="s",
        num_cores=NC, num_subcores=NS)


def _wid():
    return lax.axis_index("s") * NC + lax.axis_index("c")


def _sc_gather_user(uidx2, iidx2, hist_t, ucat_t, et_t):
    nrow, _ = uidx2.shape
    B = nrow * L
    nb = B // NW
    nch = nb // L

    @functools.partial(
        pl.kernel,
        out_type=(jax.ShapeDtypeStruct((B, L), jnp.int32),
                  jax.ShapeDtypeStruct((B, L), jnp.float32),
                  jax.ShapeDtypeStruct((B, L), jnp.float32)),
        mesh=_sc_mesh(),
        scratch_types=[pltpu.VMEM((nch, L), jnp.int32),
                       pltpu.VMEM((nch, L), jnp.int32),
                       pltpu.VMEM((L, L), jnp.int32),
                       pltpu.VMEM((L, L), jnp.float32),
                       pltpu.VMEM((L, L), jnp.float32),
                       pltpu.SemaphoreType.DMA],
    )
    def k(uidx_h, iidx_h, hist_h, ucat_h, et_h,
          hist_o, uc_o, et_o, uv, iv, hv, ucv, etv, sem):
        w = _wid()
        pltpu.sync_copy(uidx_h.at[pl.ds(w * nch, nch)], uv)
        pltpu.sync_copy(iidx_h.at[pl.ds(w * nch, nch)], iv)
        for c in range(nch):
            base = w * nb + c * L
            cps = [pltpu.async_copy(hist_h.at[uv.at[c]], hv, sem),
                   pltpu.async_copy(ucat_h.at[uv.at[c]], ucv, sem),
                   pltpu.async_copy(et_h.at[iv.at[c]], etv, sem)]
            for cp in cps:
                cp.wait()
            pltpu.sync_copy(hv, hist_o.at[pl.ds(base, L)])
            pltpu.sync_copy(ucv, uc_o.at[pl.ds(base, L)])
            pltpu.sync_copy(etv, et_o.at[pl.ds(base, L)])

    return k(uidx2, iidx2, hist_t, ucat_t, et_t)


def _sc_gather_items(ridx2, comb_t):
    nrow, _ = ridx2.shape
    T = nrow * L
    nb = T // NW
    nch = nb // L
    G = 5
    D = comb_t.shape[1]

    @functools.partial(
        pl.kernel,
        out_type=jax.ShapeDtypeStruct((T, D), jnp.int32),
        mesh=_sc_mesh(),
        scratch_types=[pltpu.VMEM((nch, L), jnp.int32),
                       pltpu.VMEM((G * L, D), jnp.int32),
                       pltpu.SemaphoreType.DMA],
    )
    def k(ridx_h, comb_h, comb_o, idxv, buf, sem):
        w = _wid()
        pltpu.sync_copy(ridx_h.at[pl.ds(w * nch, nch)], idxv)
        wbase = w * nb

        def grp(g, carry):
            cps = []
            for t in range(G):
                cps.append(pltpu.async_copy(
                    comb_h.at[idxv.at[g * G + t]],
                    buf.at[pl.ds(t * L, L)], sem))
            for cp in cps:
                cp.wait()
            pltpu.sync_copy(buf, comb_o.at[pl.ds(wbase + g * (G * L),
                                                 G * L)])
            return carry
        lax.fori_loop(0, nch // G, grp, 0)

    return k(ridx2, comb_t)


def _tc_maxlen_body(rT_ref, out_ref, *, PAD):
    r = rT_ref[...]
    lens = jnp.sum((r != PAD).astype(jnp.int32), axis=0)
    bm = jnp.max(lens)

    @pl.when(pl.program_id(0) == 0)
    def _():
        out_ref[0, 0] = 1

    out_ref[0, 0] = jnp.maximum(out_ref[0, 0], bm)


def _tc_maxlen(histT, PAD):
    H, B = histT.shape
    BB = 2048
    return pl.pallas_call(
        functools.partial(_tc_maxlen_body, PAD=PAD),
        grid=(B // BB,),
        in_specs=[pl.BlockSpec((H, BB), lambda i: (0, i))],
        out_specs=pl.BlockSpec(memory_space=pltpu.SMEM),
        out_shape=jax.ShapeDtypeStruct((1, 1), jnp.int32),
        compiler_params=pltpu.CompilerParams(
            dimension_semantics=("arbitrary",)),
    )(histT)


def _tc_main_body(rT_ref, item_ref, ucat_ref, et_ref, comb_ref,
                  Wx_ref, W1u_ref, b1_ref, W2_ref, b2_ref, W3_ref, b3_ref,
                  g1x_ref, g2_ref, wl_ref, bl_ref, ml_ref, out_ref,
                  *, H, PAD, BB):
    F = et_ref.shape[1]
    half = F // 2
    T = BB * H
    rT = rT_ref[...]
    ucat = ucat_ref[...]
    umlp = ucat[:, half:]
    comb = comb_ref[...]
    acat3 = lax.bitcast_convert_type(
        jnp.bitwise_and(comb, jnp.int32(-65536)), jnp.float32)
    eh3 = lax.bitcast_convert_type(
        jnp.left_shift(comb, 16), jnp.float32)
    acat3b = acat3.astype(jnp.bfloat16)
    zu = jnp.dot(umlp, W1u_ref[...],
                 preferred_element_type=jnp.float32) + b1_ref[...]
    x1 = jnp.dot(acat3b.reshape(T, F), Wx_ref[...],
                 preferred_element_type=jnp.float32)
    h1 = jnp.maximum(x1.reshape(H, BB, x1.shape[1]) + zu[None, :, :], 0.0)
    h1b = h1.reshape(T, x1.shape[1]).astype(jnp.bfloat16)
    h2 = jnp.maximum(jnp.dot(h1b, W2_ref[...],
                             preferred_element_type=jnp.float32)
                     + b2_ref[...], 0.0)
    h3 = jnp.maximum(jnp.dot(h2.astype(jnp.bfloat16), W3_ref[...],
                             preferred_element_type=jnp.float32)
                     + b3_ref[...], 0.0)
    g2 = g2_ref[0, :]
    s_mlp = jnp.sum(h3.reshape(H, BB, half) * g2[None, None, :], axis=2)
    ug1 = ucat * g1x_ref[...]
    s_gmf = jnp.sum(ug1[None, :, :] * acat3, axis=2)
    scores = (s_gmf + s_mlp) * (1.0 / math.sqrt(F))
    item = item_ref[...]
    bad = (rT == item) | (rT == PAD)
    scores = jnp.where(bad, jnp.float32(-1e9), scores)
    row = lax.broadcasted_iota(jnp.int32, (H, BB), 0)
    scores = jnp.where(row >= ml_ref[0, 0], jnp.float32(-2e9), scores)
    m = jnp.max(scores, axis=0, keepdims=True)
    e = jnp.exp(scores - m)
    w = e / jnp.sum(e, axis=0, keepdims=True)
    q = et_ref[...] * wl_ref[...]
    vdot = jnp.sum(eh3 * q[None, :, :], axis=2)
    out_ref[...] = jnp.sum(w * vdot, axis=0, keepdims=True) + bl_ref[...]


def _tc_main(histT, item_row, ucat, et, comb3,
             Wx, W1u, b1, W2, b2, W3, b3, g1x, g2row, wl2d, bl2d,
             maxlen, H, PAD):
    _, B = histT.shape
    F = et.shape[1]
    D1 = Wx.shape[1]
    BB = 128
    grid = (B // BB,)
    full = lambda s: pl.BlockSpec(s, lambda i: tuple(0 for _ in s))
    return pl.pallas_call(
        functools.partial(_tc_main_body, H=H, PAD=PAD, BB=BB),
        grid=grid,
        in_specs=[
            pl.BlockSpec((H, BB), lambda i: (0, i)),
            pl.BlockSpec((1, BB), lambda i: (0, i)),
            pl.BlockSpec((BB, F), lambda i: (i, 0)),
            pl.BlockSpec((BB, F), lambda i: (i, 0)),
            pl.BlockSpec((H, BB, F), lambda i: (0, i, 0)),
            full((F, D1)),
            full(W1u.shape),
            full((1, D1)),
            full(W2.shape),
            full((1, W2.shape[1])),
            full(W3.shape),
            full((1, W3.shape[1])),
            full((1, F)),
            full((1, F // 2)),
            full((1, F)),
            full((1, 1)),
            pl.BlockSpec(memory_space=pltpu.SMEM),
        ],
        out_specs=pl.BlockSpec((1, BB), lambda i: (0, i)),
        out_shape=jax.ShapeDtypeStruct((1, B), jnp.float32),
        compiler_params=pltpu.CompilerParams(
            dimension_semantics=("arbitrary",)),
    )(histT, item_row, ucat, et, comb3,
      Wx, W1u, b1.reshape(1, -1), W2, b2.reshape(1, -1), W3,
      b3.reshape(1, -1), g1x, g2row, wl2d, bl2d, maxlen)


def kernel(user_idx, item_idx, user_hist, embed_global, embed_target,
           embed_hist, aff_user_gmf, aff_item_gmf, aff_user_mlp,
           aff_item_mlp, W1, b1, W2, b2, W3, b3, W_logit, b_logit):
    B = user_idx.shape[0]
    H = user_hist.shape[1]
    F = embed_global.shape[0]
    PAD = embed_target.shape[0] - 1
    half = F // 2

    user_idx = user_idx.astype(jnp.int32)
    item_idx = item_idx.astype(jnp.int32)
    hist_t = jnp.pad(user_hist.astype(jnp.int32), ((0, 0), (0, L - H)),
                     constant_values=PAD)
    ucat_t = jnp.concatenate([aff_user_gmf, aff_user_mlp], axis=1)
    acat_f = jnp.concatenate([aff_item_gmf, aff_item_mlp], axis=1)
    a_bits = lax.bitcast_convert_type(
        acat_f.astype(jnp.bfloat16), jnp.uint16).astype(jnp.uint32) << 16
    b_bits = lax.bitcast_convert_type(
        embed_hist.astype(jnp.bfloat16), jnp.uint16).astype(jnp.uint32)
    comb_t = lax.bitcast_convert_type(a_bits | b_bits, jnp.int32)
    Wx = jnp.concatenate([jnp.zeros_like(W1[half:, :]), W1[half:, :]],
                         axis=0).astype(jnp.bfloat16)
    W1u = W1[:half, :]
    g1x = jnp.concatenate([embed_global[:half],
                           jnp.zeros((half,), jnp.float32)]).reshape(1, -1)
    g2row = embed_global[half:].reshape(1, -1)

    uidx2 = user_idx.reshape(-1, 128)
    iidx2 = item_idx.reshape(-1, 128)

    histB, ucat, et = _sc_gather_user(uidx2, iidx2, hist_t, ucat_t,
                                      embed_target)

    histT = histB[:, :H].T
    ridx2 = histT.reshape(-1, 128)
    comb = _sc_gather_items(ridx2, comb_t)
    comb3 = comb.reshape(H, B, F)

    maxlen = _tc_maxlen(histT, PAD)

    logit2 = _tc_main(histT, item_idx.reshape(1, -1), ucat, et, comb3,
                      Wx, W1u, b1, W2.astype(jnp.bfloat16), b2,
                      W3.astype(jnp.bfloat16), b3, g1x, g2row,
                      W_logit.reshape(1, -1), b_logit.reshape(1, 1),
                      maxlen, H, PAD)
    return logit2[0]


# speedup vs baseline: 20.6675x; 1.1212x over previous
import functools
import math

import jax
import jax.numpy as jnp
from jax import lax
from jax.experimental import pallas as pl
from jax.experimental.pallas import tpu as pltpu
from jax.experimental.pallas import tpu_sc as plsc

NC, NS = 2, 16
NW = NC * NS
L = 128


def _sc_mesh():
    return plsc.VectorSubcoreMesh(
        core_axis_name="c", subcore_axis_name="s",
        num_cores=NC, num_subcores=NS)


def _wid():
    return lax.axis_index("s") * NC + lax.axis_index("c")


def _sc_gather_user(uidx2, iidx2, hist_t, ucat_t, et_t):
    nrow, _ = uidx2.shape
    B = nrow * L
    nb = B // NW
    nch = nb // L

    @functools.partial(
        pl.kernel,
        out_type=(jax.ShapeDtypeStruct((B, L), jnp.int32),
                  jax.ShapeDtypeStruct((B, L), jnp.float32),
                  jax.ShapeDtypeStruct((B, L), jnp.float32)),
        mesh=_sc_mesh(),
        scratch_types=[pltpu.VMEM((nch, L), jnp.int32),
                       pltpu.VMEM((nch, L), jnp.int32),
                       pltpu.VMEM((L, L), jnp.int32),
                       pltpu.VMEM((L, L), jnp.float32),
                       pltpu.VMEM((L, L), jnp.float32),
                       pltpu.SemaphoreType.DMA],
    )
    def k(uidx_h, iidx_h, hist_h, ucat_h, et_h,
          hist_o, uc_o, et_o, uv, iv, hv, ucv, etv, sem):
        w = _wid()
        pltpu.sync_copy(uidx_h.at[pl.ds(w * nch, nch)], uv)
        pltpu.sync_copy(iidx_h.at[pl.ds(w * nch, nch)], iv)
        for c in range(nch):
            base = w * nb + c * L
            cps = [pltpu.async_copy(hist_h.at[uv.at[c]], hv, sem),
                   pltpu.async_copy(ucat_h.at[uv.at[c]], ucv, sem),
                   pltpu.async_copy(et_h.at[iv.at[c]], etv, sem)]
            for cp in cps:
                cp.wait()
            pltpu.sync_copy(hv, hist_o.at[pl.ds(base, L)])
            pltpu.sync_copy(ucv, uc_o.at[pl.ds(base, L)])
            pltpu.sync_copy(etv, et_o.at[pl.ds(base, L)])

    return k(uidx2, iidx2, hist_t, ucat_t, et_t)


def _sc_gather_items(ridx1, comb_t):
    T = ridx1.shape[0]
    nb = T // NW
    nch = nb // L
    G = 5 if nch % 5 == 0 else 4
    D = comb_t.shape[1]

    @functools.partial(
        pl.kernel,
        out_type=jax.ShapeDtypeStruct((T, D), jnp.int32),
        mesh=_sc_mesh(),
        scratch_types=[pltpu.VMEM((nb,), jnp.int32),
                       pltpu.VMEM((G * L, D), jnp.int32),
                       pltpu.SemaphoreType.DMA],
    )
    def k(ridx_h, comb_h, comb_o, idxv, buf, sem):
        w = _wid()
        pltpu.sync_copy(ridx_h.at[pl.ds(w * nb, nb)], idxv)
        wbase = w * nb

        def grp(g, carry):
            cps = []
            for t in range(G):
                cps.append(pltpu.async_copy(
                    comb_h.at[idxv.at[pl.ds((g * G + t) * L, L)]],
                    buf.at[pl.ds(t * L, L)], sem))
            for cp in cps:
                cp.wait()
            pltpu.sync_copy(buf, comb_o.at[pl.ds(wbase + g * (G * L),
                                                 G * L)])
            return carry
        lax.fori_loop(0, nch // G, grp, 0)

    return k(ridx1, comb_t)


def _tc_maxlen_body(rT_ref, out_ref, *, PAD):
    r = rT_ref[...]
    lens = jnp.sum((r != PAD).astype(jnp.int32), axis=0)
    bm = jnp.max(lens)

    @pl.when(pl.program_id(0) == 0)
    def _():
        out_ref[0, 0] = 1

    out_ref[0, 0] = jnp.maximum(out_ref[0, 0], bm)


def _tc_maxlen(histT, PAD):
    H, B = histT.shape
    BB = 2048
    return pl.pallas_call(
        functools.partial(_tc_maxlen_body, PAD=PAD),
        grid=(B // BB,),
        in_specs=[pl.BlockSpec((H, BB), lambda i: (0, i))],
        out_specs=pl.BlockSpec(memory_space=pltpu.SMEM),
        out_shape=jax.ShapeDtypeStruct((1, 1), jnp.int32),
        compiler_params=pltpu.CompilerParams(
            dimension_semantics=("arbitrary",)),
    )(histT)


def _tc_main_body(rT_ref, item_ref, ucat_ref, et_ref, comb_ref,
                  Wx_ref, W1u_ref, b1_ref, W2_ref, b2_ref, W3_ref, b3_ref,
                  g1x_ref, g2_ref, wl_ref, bl_ref, ml_ref, out_ref,
                  *, H, PAD, BB):
    F = et_ref.shape[1]
    half = F // 2
    T = BB * H
    rT = rT_ref[...]
    ucat = ucat_ref[...]
    umlp = ucat[:, half:]
    comb = comb_ref[...]
    acat3 = lax.bitcast_convert_type(
        jnp.bitwise_and(comb, jnp.int32(-65536)), jnp.float32)
    eh3 = lax.bitcast_convert_type(
        jnp.left_shift(comb, 16), jnp.float32)
    acat3b = acat3.astype(jnp.bfloat16)
    zu = jnp.dot(umlp, W1u_ref[...],
                 preferred_element_type=jnp.float32) + b1_ref[...]
    x1 = jnp.dot(acat3b.reshape(T, F), Wx_ref[...],
                 preferred_element_type=jnp.float32)
    h1 = jnp.maximum(x1.reshape(H, BB, x1.shape[1]) + zu[None, :, :], 0.0)
    h1b = h1.reshape(T, x1.shape[1]).astype(jnp.bfloat16)
    h2 = jnp.maximum(jnp.dot(h1b, W2_ref[...],
                             preferred_element_type=jnp.float32)
                     + b2_ref[...], 0.0)
    h3 = jnp.maximum(jnp.dot(h2.astype(jnp.bfloat16), W3_ref[...],
                             preferred_element_type=jnp.float32)
                     + b3_ref[...], 0.0)
    g2 = g2_ref[0, :]
    s_mlp = jnp.sum(h3.reshape(H, BB, half) * g2[None, None, :], axis=2)
    ug1 = ucat * g1x_ref[...]
    s_gmf = jnp.sum(ug1[None, :, :] * acat3, axis=2)
    scores = (s_gmf + s_mlp) * (1.0 / math.sqrt(F))
    item = item_ref[...]
    bad = (rT == item) | (rT == PAD)
    scores = jnp.where(bad, jnp.float32(-1e9), scores)
    row = lax.broadcasted_iota(jnp.int32, (H, BB), 0)
    scores = jnp.where(row >= ml_ref[0, 0], jnp.float32(-2e9), scores)
    m = jnp.max(scores, axis=0, keepdims=True)
    e = jnp.exp(scores - m)
    se = jnp.sum(e, axis=0, keepdims=True)
    q = et_ref[...] * wl_ref[...]
    vdot = jnp.sum(eh3 * q[None, :, :], axis=2)
    out_ref[...] = (jnp.sum(e * vdot, axis=0, keepdims=True) / se
                    + bl_ref[...])


def _tc_main(histT, item_row, ucat, et, comb3,
             Wx, W1u, b1, W2, b2, W3, b3, g1x, g2row, wl2d, bl2d,
             maxlen, H, PAD):
    _, B = histT.shape
    F = et.shape[1]
    D1 = Wx.shape[1]
    BB = 128
    grid = (B // BB,)
    full = lambda s: pl.BlockSpec(s, lambda i: tuple(0 for _ in s))
    return pl.pallas_call(
        functools.partial(_tc_main_body, H=H, PAD=PAD, BB=BB),
        grid=grid,
        in_specs=[
            pl.BlockSpec((H, BB), lambda i: (0, i)),
            pl.BlockSpec((1, BB), lambda i: (0, i)),
            pl.BlockSpec((BB, F), lambda i: (i, 0)),
            pl.BlockSpec((BB, F), lambda i: (i, 0)),
            pl.BlockSpec((H, BB, F), lambda i: (0, i, 0)),
            full((F, D1)),
            full(W1u.shape),
            full((1, D1)),
            full(W2.shape),
            full((1, W2.shape[1])),
            full(W3.shape),
            full((1, W3.shape[1])),
            full((1, F)),
            full((1, F // 2)),
            full((1, F)),
            full((1, 1)),
            pl.BlockSpec(memory_space=pltpu.SMEM),
        ],
        out_specs=pl.BlockSpec((1, BB), lambda i: (0, i)),
        out_shape=jax.ShapeDtypeStruct((1, B), jnp.float32),
        compiler_params=pltpu.CompilerParams(
            dimension_semantics=("arbitrary",)),
    )(histT, item_row, ucat, et, comb3,
      Wx, W1u, b1.reshape(1, -1), W2, b2.reshape(1, -1), W3,
      b3.reshape(1, -1), g1x, g2row, wl2d, bl2d, maxlen)


def kernel(user_idx, item_idx, user_hist, embed_global, embed_target,
           embed_hist, aff_user_gmf, aff_item_gmf, aff_user_mlp,
           aff_item_mlp, W1, b1, W2, b2, W3, b3, W_logit, b_logit):
    B = user_idx.shape[0]
    H = user_hist.shape[1]
    F = embed_global.shape[0]
    PAD = embed_target.shape[0] - 1
    half = F // 2

    user_idx = user_idx.astype(jnp.int32)
    item_idx = item_idx.astype(jnp.int32)
    hist_t = jnp.pad(user_hist.astype(jnp.int32), ((0, 0), (0, L - H)),
                     constant_values=PAD)
    ucat_t = jnp.concatenate([aff_user_gmf, aff_user_mlp], axis=1)
    acat_f = jnp.concatenate([aff_item_gmf, aff_item_mlp], axis=1)
    a_bits = lax.bitcast_convert_type(
        acat_f.astype(jnp.bfloat16), jnp.uint16).astype(jnp.uint32) << 16
    b_bits = lax.bitcast_convert_type(
        embed_hist.astype(jnp.bfloat16), jnp.uint16).astype(jnp.uint32)
    comb_t = lax.bitcast_convert_type(a_bits | b_bits, jnp.int32)
    Wx = jnp.concatenate([jnp.zeros_like(W1[half:, :]), W1[half:, :]],
                         axis=0).astype(jnp.bfloat16)
    W1u = W1[:half, :]
    g1x = jnp.concatenate([embed_global[:half],
                           jnp.zeros((half,), jnp.float32)]).reshape(1, -1)
    g2row = embed_global[half:].reshape(1, -1)

    uidx2 = user_idx.reshape(-1, 128)
    iidx2 = item_idx.reshape(-1, 128)

    histB, ucat, et = _sc_gather_user(uidx2, iidx2, hist_t, ucat_t,
                                      embed_target)

    histT = histB[:, :H].T
    maxlen = _tc_maxlen(histT, PAD)
    item_row = item_idx.reshape(1, -1)

    NCHUNK = 2
    CB = B // NCHUNK
    outs = []
    for j in range(NCHUNK):
        sl = slice(j * CB, (j + 1) * CB)
        histT_c = histT[:, sl]
        comb_c = _sc_gather_items(histT_c.reshape(-1), comb_t)
        outs.append(_tc_main(
            histT_c, item_row[:, sl], ucat[sl], et[sl],
            comb_c.reshape(H, CB, F),
            Wx, W1u, b1, W2.astype(jnp.bfloat16), b2,
            W3.astype(jnp.bfloat16), b3, g1x, g2row,
            W_logit.reshape(1, -1), b_logit.reshape(1, 1),
            maxlen, H, PAD))
    return jnp.concatenate(outs, axis=1)[0]


# speedup vs baseline: 22.1878x; 1.0736x over previous
import functools
import math

import jax
import jax.numpy as jnp
from jax import lax
from jax.experimental import pallas as pl
from jax.experimental.pallas import tpu as pltpu
from jax.experimental.pallas import tpu_sc as plsc

NC, NS = 2, 16
NW = NC * NS
L = 128


def _sc_mesh():
    return plsc.VectorSubcoreMesh(
        core_axis_name="c", subcore_axis_name="s",
        num_cores=NC, num_subcores=NS)


def _wid():
    return lax.axis_index("s") * NC + lax.axis_index("c")


def _sc_gather_user(uidx2, iidx2, hist_t, ucat_t, et_t):
    nrow, _ = uidx2.shape
    B = nrow * L
    nb = B // NW
    nch = nb // L

    @functools.partial(
        pl.kernel,
        out_type=(jax.ShapeDtypeStruct((B, L), jnp.int32),
                  jax.ShapeDtypeStruct((B, L), jnp.float32),
                  jax.ShapeDtypeStruct((B, L), jnp.float32)),
        mesh=_sc_mesh(),
        scratch_types=[pltpu.VMEM((nch, L), jnp.int32),
                       pltpu.VMEM((nch, L), jnp.int32),
                       pltpu.VMEM((L, L), jnp.int32),
                       pltpu.VMEM((L, L), jnp.float32),
                       pltpu.VMEM((L, L), jnp.float32),
                       pltpu.SemaphoreType.DMA],
    )
    def k(uidx_h, iidx_h, hist_h, ucat_h, et_h,
          hist_o, uc_o, et_o, uv, iv, hv, ucv, etv, sem):
        w = _wid()
        pltpu.sync_copy(uidx_h.at[pl.ds(w * nch, nch)], uv)
        pltpu.sync_copy(iidx_h.at[pl.ds(w * nch, nch)], iv)
        for c in range(nch):
            base = w * nb + c * L
            cps = [pltpu.async_copy(hist_h.at[uv.at[c]], hv, sem),
                   pltpu.async_copy(ucat_h.at[uv.at[c]], ucv, sem),
                   pltpu.async_copy(et_h.at[iv.at[c]], etv, sem)]
            for cp in cps:
                cp.wait()
            pltpu.sync_copy(hv, hist_o.at[pl.ds(base, L)])
            pltpu.sync_copy(ucv, uc_o.at[pl.ds(base, L)])
            pltpu.sync_copy(etv, et_o.at[pl.ds(base, L)])

    return k(uidx2, iidx2, hist_t, ucat_t, et_t)


def _sc_gather_items(ridx1, comb_t):
    T = ridx1.shape[0]
    nb = T // NW
    nch = nb // L
    G = 5 if nch % 5 == 0 else 4
    D = comb_t.shape[1]

    @functools.partial(
        pl.kernel,
        out_type=jax.ShapeDtypeStruct((T, D), jnp.int32),
        mesh=_sc_mesh(),
        scratch_types=[pltpu.VMEM((nb,), jnp.int32),
                       pltpu.VMEM((G * L, D), jnp.int32),
                       pltpu.SemaphoreType.DMA],
    )
    def k(ridx_h, comb_h, comb_o, idxv, buf, sem):
        w = _wid()
        pltpu.sync_copy(ridx_h.at[pl.ds(w * nb, nb)], idxv)
        wbase = w * nb

        def grp(g, carry):
            cps = []
            for t in range(G):
                cps.append(pltpu.async_copy(
                    comb_h.at[idxv.at[pl.ds((g * G + t) * L, L)]],
                    buf.at[pl.ds(t * L, L)], sem))
            for cp in cps:
                cp.wait()
            pltpu.sync_copy(buf, comb_o.at[pl.ds(wbase + g * (G * L),
                                                 G * L)])
            return carry
        lax.fori_loop(0, nch // G, grp, 0)

    return k(ridx1, comb_t)


def _tc_maxlen_body(rT_ref, out_ref, *, PAD):
    r = rT_ref[...]
    lens = jnp.sum((r != PAD).astype(jnp.int32), axis=0)
    bm = jnp.max(lens)

    @pl.when(pl.program_id(0) == 0)
    def _():
        out_ref[0, 0] = 1

    out_ref[0, 0] = jnp.maximum(out_ref[0, 0], bm)


def _tc_maxlen(histT, PAD):
    H, B = histT.shape
    BB = 2048
    return pl.pallas_call(
        functools.partial(_tc_maxlen_body, PAD=PAD),
        grid=(B // BB,),
        in_specs=[pl.BlockSpec((H, BB), lambda i: (0, i))],
        out_specs=pl.BlockSpec(memory_space=pltpu.SMEM),
        out_shape=jax.ShapeDtypeStruct((1, 1), jnp.int32),
        compiler_params=pltpu.CompilerParams(
            dimension_semantics=("arbitrary",)),
    )(histT)


def _tc_main_body(rT_ref, item_ref, ucat_ref, et_ref, comb_ref,
                  Wx_ref, W1u_ref, b1_ref, W2_ref, b2_ref, W3_ref, b3_ref,
                  g1x_ref, g2_ref, wl_ref, bl_ref, ml_ref, out_ref,
                  *, H, PAD, BB):
    F = et_ref.shape[1]
    half = F // 2
    T = BB * H
    rT = rT_ref[...]
    ucat = ucat_ref[...]
    umlp = ucat[:, half:]
    comb = comb_ref[...]
    acat3 = lax.bitcast_convert_type(
        jnp.bitwise_and(comb, jnp.int32(-65536)), jnp.float32)
    eh3 = lax.bitcast_convert_type(
        jnp.left_shift(comb, 16), jnp.float32)
    acat3b = acat3.astype(jnp.bfloat16)
    zu = jnp.dot(umlp, W1u_ref[...],
                 preferred_element_type=jnp.float32) + b1_ref[...]
    x1 = jnp.dot(acat3b.reshape(T, F), Wx_ref[...],
                 preferred_element_type=jnp.float32)
    h1 = jnp.maximum(x1.reshape(H, BB, x1.shape[1]) + zu[None, :, :], 0.0)
    h1b = h1.reshape(T, x1.shape[1]).astype(jnp.bfloat16)
    h2 = jnp.maximum(jnp.dot(h1b, W2_ref[...],
                             preferred_element_type=jnp.float32)
                     + b2_ref[...], 0.0)
    h3 = jnp.maximum(jnp.dot(h2.astype(jnp.bfloat16), W3_ref[...],
                             preferred_element_type=jnp.float32)
                     + b3_ref[...], 0.0)
    g2 = g2_ref[0, :]
    s_mlp = jnp.sum(h3.reshape(H, BB, half) * g2[None, None, :], axis=2)
    ug1 = ucat * g1x_ref[...]
    s_gmf = jnp.sum(ug1[None, :, :] * acat3, axis=2)
    scores = (s_gmf + s_mlp) * (1.0 / math.sqrt(F))
    item = item_ref[...]
    bad = (rT == item) | (rT == PAD)
    scores = jnp.where(bad, jnp.float32(-1e9), scores)
    row = lax.broadcasted_iota(jnp.int32, (H, BB), 0)
    scores = jnp.where(row >= ml_ref[0, 0], jnp.float32(-2e9), scores)
    m = jnp.max(scores, axis=0, keepdims=True)
    e = jnp.exp(scores - m)
    se = jnp.sum(e, axis=0, keepdims=True)
    q = et_ref[...] * wl_ref[...]
    vdot = jnp.sum(eh3 * q[None, :, :], axis=2)
    out_ref[...] = (jnp.sum(e * vdot, axis=0, keepdims=True) / se
                    + bl_ref[...])


def _tc_main(histT, item_row, ucat, et, comb3,
             Wx, W1u, b1, W2, b2, W3, b3, g1x, g2row, wl2d, bl2d,
             maxlen, H, PAD):
    _, B = histT.shape
    F = et.shape[1]
    D1 = Wx.shape[1]
    BB = 128
    grid = (B // BB,)
    full = lambda s: pl.BlockSpec(s, lambda i: tuple(0 for _ in s))
    return pl.pallas_call(
        functools.partial(_tc_main_body, H=H, PAD=PAD, BB=BB),
        grid=grid,
        in_specs=[
            pl.BlockSpec((H, BB), lambda i: (0, i)),
            pl.BlockSpec((1, BB), lambda i: (0, i)),
            pl.BlockSpec((BB, F), lambda i: (i, 0)),
            pl.BlockSpec((BB, F), lambda i: (i, 0)),
            pl.BlockSpec((H, BB, F), lambda i: (0, i, 0)),
            full((F, D1)),
            full(W1u.shape),
            full((1, D1)),
            full(W2.shape),
            full((1, W2.shape[1])),
            full(W3.shape),
            full((1, W3.shape[1])),
            full((1, F)),
            full((1, F // 2)),
            full((1, F)),
            full((1, 1)),
            pl.BlockSpec(memory_space=pltpu.SMEM),
        ],
        out_specs=pl.BlockSpec((1, BB), lambda i: (0, i)),
        out_shape=jax.ShapeDtypeStruct((1, B), jnp.float32),
        compiler_params=pltpu.CompilerParams(
            dimension_semantics=("arbitrary",)),
    )(histT, item_row, ucat, et, comb3,
      Wx, W1u, b1.reshape(1, -1), W2, b2.reshape(1, -1), W3,
      b3.reshape(1, -1), g1x, g2row, wl2d, bl2d, maxlen)


def kernel(user_idx, item_idx, user_hist, embed_global, embed_target,
           embed_hist, aff_user_gmf, aff_item_gmf, aff_user_mlp,
           aff_item_mlp, W1, b1, W2, b2, W3, b3, W_logit, b_logit):
    B = user_idx.shape[0]
    H = user_hist.shape[1]
    F = embed_global.shape[0]
    PAD = embed_target.shape[0] - 1
    half = F // 2

    user_idx = user_idx.astype(jnp.int32)
    item_idx = item_idx.astype(jnp.int32)
    hist_t = jnp.pad(user_hist.astype(jnp.int32), ((0, 0), (0, L - H)),
                     constant_values=PAD)
    ucat_t = jnp.concatenate([aff_user_gmf, aff_user_mlp], axis=1)
    acat_f = jnp.concatenate([aff_item_gmf, aff_item_mlp], axis=1)
    a_bits = lax.bitcast_convert_type(
        acat_f.astype(jnp.bfloat16), jnp.uint16).astype(jnp.uint32) << 16
    b_bits = lax.bitcast_convert_type(
        embed_hist.astype(jnp.bfloat16), jnp.uint16).astype(jnp.uint32)
    comb_t = lax.bitcast_convert_type(a_bits | b_bits, jnp.int32)
    Wx = jnp.concatenate([jnp.zeros_like(W1[half:, :]), W1[half:, :]],
                         axis=0).astype(jnp.bfloat16)
    W1u = W1[:half, :]
    g1x = jnp.concatenate([embed_global[:half],
                           jnp.zeros((half,), jnp.float32)]).reshape(1, -1)
    g2row = embed_global[half:].reshape(1, -1)

    uidx2 = user_idx.reshape(-1, 128)
    iidx2 = item_idx.reshape(-1, 128)

    histB, ucat, et = _sc_gather_user(uidx2, iidx2, hist_t, ucat_t,
                                      embed_target)

    histT = histB[:, :H].T
    maxlen = _tc_maxlen(histT, PAD)
    item_row = item_idx.reshape(1, -1)

    NCHUNK = 4
    CB = B // NCHUNK
    outs = []
    for j in range(NCHUNK):
        sl = slice(j * CB, (j + 1) * CB)
        histT_c = histT[:, sl]
        comb_c = _sc_gather_items(histT_c.reshape(-1), comb_t)
        outs.append(_tc_main(
            histT_c, item_row[:, sl], ucat[sl], et[sl],
            comb_c.reshape(H, CB, F),
            Wx, W1u, b1, W2.astype(jnp.bfloat16), b2,
            W3.astype(jnp.bfloat16), b3, g1x, g2row,
            W_logit.reshape(1, -1), b_logit.reshape(1, 1),
            maxlen, H, PAD))
    return jnp.concatenate(outs, axis=1)[0]


# speedup vs baseline: 23.4204x; 1.0556x over previous
import functools
import math

import jax
import jax.numpy as jnp
from jax import lax
from jax.experimental import pallas as pl
from jax.experimental.pallas import tpu as pltpu
from jax.experimental.pallas import tpu_sc as plsc

NC, NS = 2, 16
NW = NC * NS
L = 128


def _sc_mesh():
    return plsc.VectorSubcoreMesh(
        core_axis_name="c", subcore_axis_name="s",
        num_cores=NC, num_subcores=NS)


def _wid():
    return lax.axis_index("s") * NC + lax.axis_index("c")


def _sc_gather_user(uidx2, iidx2, hist_t, ucat_t, et_t):
    nrow, _ = uidx2.shape
    B = nrow * L
    nb = B // NW
    nch = nb // L

    @functools.partial(
        pl.kernel,
        out_type=(jax.ShapeDtypeStruct((B, L), jnp.int32),
                  jax.ShapeDtypeStruct((B, L), jnp.float32),
                  jax.ShapeDtypeStruct((B, L), jnp.float32)),
        mesh=_sc_mesh(),
        scratch_types=[pltpu.VMEM((nch, L), jnp.int32),
                       pltpu.VMEM((nch, L), jnp.int32),
                       pltpu.VMEM((L, L), jnp.int32),
                       pltpu.VMEM((L, L), jnp.float32),
                       pltpu.VMEM((L, L), jnp.float32),
                       pltpu.SemaphoreType.DMA],
    )
    def k(uidx_h, iidx_h, hist_h, ucat_h, et_h,
          hist_o, uc_o, et_o, uv, iv, hv, ucv, etv, sem):
        w = _wid()
        pltpu.sync_copy(uidx_h.at[pl.ds(w * nch, nch)], uv)
        pltpu.sync_copy(iidx_h.at[pl.ds(w * nch, nch)], iv)
        for c in range(nch):
            base = w * nb + c * L
            cps = [pltpu.async_copy(hist_h.at[uv.at[c]], hv, sem),
                   pltpu.async_copy(ucat_h.at[uv.at[c]], ucv, sem),
                   pltpu.async_copy(et_h.at[iv.at[c]], etv, sem)]
            for cp in cps:
                cp.wait()
            pltpu.sync_copy(hv, hist_o.at[pl.ds(base, L)])
            pltpu.sync_copy(ucv, uc_o.at[pl.ds(base, L)])
            pltpu.sync_copy(etv, et_o.at[pl.ds(base, L)])

    return k(uidx2, iidx2, hist_t, ucat_t, et_t)


def _sc_gather_items(ridx1, comb_t):
    T = ridx1.shape[0]
    nb = T // NW
    nch = nb // L
    G = 5 if nch % 5 == 0 else 4
    D = comb_t.shape[1]

    @functools.partial(
        pl.kernel,
        out_type=jax.ShapeDtypeStruct((T, D), jnp.int32),
        mesh=_sc_mesh(),
        scratch_types=[pltpu.VMEM((nb,), jnp.int32),
                       pltpu.VMEM((G * L, D), jnp.int32),
                       pltpu.SemaphoreType.DMA],
    )
    def k(ridx_h, comb_h, comb_o, idxv, buf, sem):
        w = _wid()
        pltpu.sync_copy(ridx_h.at[pl.ds(w * nb, nb)], idxv)
        wbase = w * nb

        def grp(g, carry):
            cps = []
            for t in range(G):
                cps.append(pltpu.async_copy(
                    comb_h.at[idxv.at[pl.ds((g * G + t) * L, L)]],
                    buf.at[pl.ds(t * L, L)], sem))
            for cp in cps:
                cp.wait()
            pltpu.sync_copy(buf, comb_o.at[pl.ds(wbase + g * (G * L),
                                                 G * L)])
            return carry
        lax.fori_loop(0, nch // G, grp, 0)

    return k(ridx1, comb_t)


def _tc_maxlen_body(rT_ref, out_ref, *, PAD):
    r = rT_ref[...]
    lens = jnp.sum((r != PAD).astype(jnp.int32), axis=0)
    bm = jnp.max(lens)

    @pl.when(pl.program_id(0) == 0)
    def _():
        out_ref[0, 0] = 1

    out_ref[0, 0] = jnp.maximum(out_ref[0, 0], bm)


def _tc_maxlen(histT, PAD):
    H, B = histT.shape
    BB = 2048
    return pl.pallas_call(
        functools.partial(_tc_maxlen_body, PAD=PAD),
        grid=(B // BB,),
        in_specs=[pl.BlockSpec((H, BB), lambda i: (0, i))],
        out_specs=pl.BlockSpec(memory_space=pltpu.SMEM),
        out_shape=jax.ShapeDtypeStruct((1, 1), jnp.int32),
        compiler_params=pltpu.CompilerParams(
            dimension_semantics=("arbitrary",)),
    )(histT)


def _tc_main_body(rT_ref, item_ref, ucat_ref, et_ref, comb_ref,
                  Wx_ref, W1u_ref, b1_ref, W2_ref, b2_ref, W3_ref, b3_ref,
                  g1x_ref, g2_ref, wl_ref, bl_ref, ml_ref, out_ref,
                  *, H, PAD, BB):
    F = et_ref.shape[1]
    half = F // 2
    T = BB * H
    rT = rT_ref[...]
    ucat = ucat_ref[...]
    umlp = ucat[:, half:]
    comb = comb_ref[...]
    acat3 = lax.bitcast_convert_type(
        jnp.bitwise_and(comb, jnp.int32(-65536)), jnp.float32)
    eh3 = lax.bitcast_convert_type(
        jnp.left_shift(comb, 16), jnp.float32)
    acat3b = acat3.astype(jnp.bfloat16)
    zu = jnp.dot(umlp, W1u_ref[...],
                 preferred_element_type=jnp.float32) + b1_ref[...]
    x1 = jnp.dot(acat3b.reshape(T, F), Wx_ref[...],
                 preferred_element_type=jnp.float32)
    h1 = jnp.maximum(x1.reshape(H, BB, x1.shape[1]) + zu[None, :, :], 0.0)
    h1b = h1.reshape(T, x1.shape[1]).astype(jnp.bfloat16)
    h2 = jnp.maximum(jnp.dot(h1b, W2_ref[...],
                             preferred_element_type=jnp.float32)
                     + b2_ref[...], 0.0)
    h3 = jnp.maximum(jnp.dot(h2.astype(jnp.bfloat16), W3_ref[...],
                             preferred_element_type=jnp.float32)
                     + b3_ref[...], 0.0)
    g2 = g2_ref[0, :]
    s_mlp = jnp.sum(h3.reshape(H, BB, half) * g2[None, None, :], axis=2)
    ug1 = ucat * g1x_ref[...]
    s_gmf = jnp.sum(ug1[None, :, :] * acat3, axis=2)
    scores = (s_gmf + s_mlp) * (1.0 / math.sqrt(F))
    item = item_ref[...]
    bad = (rT == item) | (rT == PAD)
    scores = jnp.where(bad, jnp.float32(-1e9), scores)
    row = lax.broadcasted_iota(jnp.int32, (H, BB), 0)
    scores = jnp.where(row >= ml_ref[0, 0], jnp.float32(-2e9), scores)
    m = jnp.max(scores, axis=0, keepdims=True)
    e = jnp.exp(scores - m)
    se = jnp.sum(e, axis=0, keepdims=True)
    q = et_ref[...] * wl_ref[...]
    vdot = jnp.sum(eh3 * q[None, :, :], axis=2)
    out_ref[...] = (jnp.sum(e * vdot, axis=0, keepdims=True) / se
                    + bl_ref[...])


def _tc_main(histT, item_row, ucat, et, comb3,
             Wx, W1u, b1, W2, b2, W3, b3, g1x, g2row, wl2d, bl2d,
             maxlen, H, PAD):
    _, B = histT.shape
    F = et.shape[1]
    D1 = Wx.shape[1]
    BB = 128
    grid = (B // BB,)
    full = lambda s: pl.BlockSpec(s, lambda i: tuple(0 for _ in s))
    return pl.pallas_call(
        functools.partial(_tc_main_body, H=H, PAD=PAD, BB=BB),
        grid=grid,
        in_specs=[
            pl.BlockSpec((H, BB), lambda i: (0, i)),
            pl.BlockSpec((1, BB), lambda i: (0, i)),
            pl.BlockSpec((BB, F), lambda i: (i, 0)),
            pl.BlockSpec((BB, F), lambda i: (i, 0)),
            pl.BlockSpec((H, BB, F), lambda i: (0, i, 0)),
            full((F, D1)),
            full(W1u.shape),
            full((1, D1)),
            full(W2.shape),
            full((1, W2.shape[1])),
            full(W3.shape),
            full((1, W3.shape[1])),
            full((1, F)),
            full((1, F // 2)),
            full((1, F)),
            full((1, 1)),
            pl.BlockSpec(memory_space=pltpu.SMEM),
        ],
        out_specs=pl.BlockSpec((1, BB), lambda i: (0, i)),
        out_shape=jax.ShapeDtypeStruct((1, B), jnp.float32),
        compiler_params=pltpu.CompilerParams(
            dimension_semantics=("arbitrary",)),
    )(histT, item_row, ucat, et, comb3,
      Wx, W1u, b1.reshape(1, -1), W2, b2.reshape(1, -1), W3,
      b3.reshape(1, -1), g1x, g2row, wl2d, bl2d, maxlen)


def kernel(user_idx, item_idx, user_hist, embed_global, embed_target,
           embed_hist, aff_user_gmf, aff_item_gmf, aff_user_mlp,
           aff_item_mlp, W1, b1, W2, b2, W3, b3, W_logit, b_logit):
    B = user_idx.shape[0]
    H = user_hist.shape[1]
    F = embed_global.shape[0]
    PAD = embed_target.shape[0] - 1
    half = F // 2

    user_idx = user_idx.astype(jnp.int32)
    item_idx = item_idx.astype(jnp.int32)
    hist_t = jnp.pad(user_hist.astype(jnp.int32), ((0, 0), (0, L - H)),
                     constant_values=PAD)
    ucat_t = jnp.concatenate([aff_user_gmf, aff_user_mlp], axis=1)
    acat_f = jnp.concatenate([aff_item_gmf, aff_item_mlp], axis=1)
    a_bits = lax.bitcast_convert_type(
        acat_f.astype(jnp.bfloat16), jnp.uint16).astype(jnp.uint32) << 16
    b_bits = lax.bitcast_convert_type(
        embed_hist.astype(jnp.bfloat16), jnp.uint16).astype(jnp.uint32)
    comb_t = lax.bitcast_convert_type(a_bits | b_bits, jnp.int32)
    Wx = jnp.concatenate([jnp.zeros_like(W1[half:, :]), W1[half:, :]],
                         axis=0).astype(jnp.bfloat16)
    W1u = W1[:half, :]
    g1x = jnp.concatenate([embed_global[:half],
                           jnp.zeros((half,), jnp.float32)]).reshape(1, -1)
    g2row = embed_global[half:].reshape(1, -1)

    uidx2 = user_idx.reshape(-1, 128)
    iidx2 = item_idx.reshape(-1, 128)

    histB, ucat, et = _sc_gather_user(uidx2, iidx2, hist_t, ucat_t,
                                      embed_target)

    histT = histB[:, :H].T
    maxlen = _tc_maxlen(histT, PAD)
    item_row = item_idx.reshape(1, -1)

    NCHUNK = 8
    CB = B // NCHUNK
    outs = []
    for j in range(NCHUNK):
        sl = slice(j * CB, (j + 1) * CB)
        histT_c = histT[:, sl]
        comb_c = _sc_gather_items(histT_c.reshape(-1), comb_t)
        outs.append(_tc_main(
            histT_c, item_row[:, sl], ucat[sl], et[sl],
            comb_c.reshape(H, CB, F),
            Wx, W1u, b1, W2.astype(jnp.bfloat16), b2,
            W3.astype(jnp.bfloat16), b3, g1x, g2row,
            W_logit.reshape(1, -1), b_logit.reshape(1, 1),
            maxlen, H, PAD))
    return jnp.concatenate(outs, axis=1)[0]


# speedup vs baseline: 23.5475x; 1.0054x over previous
import functools
import math

import jax
import jax.numpy as jnp
from jax import lax
from jax.experimental import pallas as pl
from jax.experimental.pallas import tpu as pltpu
from jax.experimental.pallas import tpu_sc as plsc

NC, NS = 2, 16
NW = NC * NS
L = 128


def _sc_mesh():
    return plsc.VectorSubcoreMesh(
        core_axis_name="c", subcore_axis_name="s",
        num_cores=NC, num_subcores=NS)


def _wid():
    return lax.axis_index("s") * NC + lax.axis_index("c")


def _sc_gather_hist(uidx2, hist_t):
    nrow, _ = uidx2.shape
    B = nrow * L
    nb = B // NW
    nch = nb // L

    @functools.partial(
        pl.kernel,
        out_type=jax.ShapeDtypeStruct((B, L), jnp.int32),
        mesh=_sc_mesh(),
        scratch_types=[pltpu.VMEM((nch, L), jnp.int32),
                       pltpu.VMEM((nch * L, L), jnp.int32),
                       pltpu.SemaphoreType.DMA],
    )
    def k(uidx_h, hist_h, hist_o, uv, hv, sem):
        w = _wid()
        pltpu.sync_copy(uidx_h.at[pl.ds(w * nch, nch)], uv)
        cps = [pltpu.async_copy(hist_h.at[uv.at[c]],
                                hv.at[pl.ds(c * L, L)], sem)
               for c in range(nch)]
        for cp in cps:
            cp.wait()
        pltpu.sync_copy(hv, hist_o.at[pl.ds(w * nb, nb)])

    return k(uidx2, hist_t)


def _sc_gather_user(uidx2, iidx2, ucat_t, et_t):
    nrow, _ = uidx2.shape
    B = nrow * L
    nb = B // NW
    nch = nb // L

    @functools.partial(
        pl.kernel,
        out_type=(jax.ShapeDtypeStruct((B, L), jnp.float32),
                  jax.ShapeDtypeStruct((B, L), jnp.float32)),
        mesh=_sc_mesh(),
        scratch_types=[pltpu.VMEM((nch, L), jnp.int32),
                       pltpu.VMEM((nch, L), jnp.int32),
                       pltpu.VMEM((L, L), jnp.float32),
                       pltpu.VMEM((L, L), jnp.float32),
                       pltpu.SemaphoreType.DMA],
    )
    def k(uidx_h, iidx_h, ucat_h, et_h,
          uc_o, et_o, uv, iv, ucv, etv, sem):
        w = _wid()
        pltpu.sync_copy(uidx_h.at[pl.ds(w * nch, nch)], uv)
        pltpu.sync_copy(iidx_h.at[pl.ds(w * nch, nch)], iv)
        for c in range(nch):
            base = w * nb + c * L
            cps = [pltpu.async_copy(ucat_h.at[uv.at[c]], ucv, sem),
                   pltpu.async_copy(et_h.at[iv.at[c]], etv, sem)]
            for cp in cps:
                cp.wait()
            pltpu.sync_copy(ucv, uc_o.at[pl.ds(base, L)])
            pltpu.sync_copy(etv, et_o.at[pl.ds(base, L)])

    return k(uidx2, iidx2, ucat_t, et_t)


def _sc_gather_items(ridx1, comb_t):
    T = ridx1.shape[0]
    nb = T // NW
    nch = nb // L
    G = 5 if nch % 5 == 0 else 4
    D = comb_t.shape[1]

    @functools.partial(
        pl.kernel,
        out_type=jax.ShapeDtypeStruct((T, D), jnp.int32),
        mesh=_sc_mesh(),
        scratch_types=[pltpu.VMEM((nb,), jnp.int32),
                       pltpu.VMEM((G * L, D), jnp.int32),
                       pltpu.SemaphoreType.DMA],
    )
    def k(ridx_h, comb_h, comb_o, idxv, buf, sem):
        w = _wid()
        pltpu.sync_copy(ridx_h.at[pl.ds(w * nb, nb)], idxv)
        wbase = w * nb

        def grp(g, carry):
            cps = []
            for t in range(G):
                cps.append(pltpu.async_copy(
                    comb_h.at[idxv.at[pl.ds((g * G + t) * L, L)]],
                    buf.at[pl.ds(t * L, L)], sem))
            for cp in cps:
                cp.wait()
            pltpu.sync_copy(buf, comb_o.at[pl.ds(wbase + g * (G * L),
                                                 G * L)])
            return carry
        lax.fori_loop(0, nch // G, grp, 0)

    return k(ridx1, comb_t)


def _tc_maxlen_body(rT_ref, out_ref, *, PAD):
    r = rT_ref[...]
    lens = jnp.sum((r != PAD).astype(jnp.int32), axis=0)
    bm = jnp.max(lens)

    @pl.when(pl.program_id(0) == 0)
    def _():
        out_ref[0, 0] = 1

    out_ref[0, 0] = jnp.maximum(out_ref[0, 0], bm)


def _tc_maxlen(histT, PAD):
    H, B = histT.shape
    BB = 2048
    return pl.pallas_call(
        functools.partial(_tc_maxlen_body, PAD=PAD),
        grid=(B // BB,),
        in_specs=[pl.BlockSpec((H, BB), lambda i: (0, i))],
        out_specs=pl.BlockSpec(memory_space=pltpu.SMEM),
        out_shape=jax.ShapeDtypeStruct((1, 1), jnp.int32),
        compiler_params=pltpu.CompilerParams(
            dimension_semantics=("arbitrary",)),
    )(histT)


def _tc_main_body(rT_ref, item_ref, ucat_ref, et_ref, comb_ref,
                  Wx_ref, W1u_ref, b1_ref, W2_ref, b2_ref, W3_ref, b3_ref,
                  g1x_ref, g2_ref, wl_ref, bl_ref, ml_ref, out_ref,
                  *, H, PAD, BB):
    F = et_ref.shape[1]
    half = F // 2
    T = BB * H
    rT = rT_ref[...]
    ucat = ucat_ref[...]
    umlp = ucat[:, half:]
    comb = comb_ref[...]
    acat3 = lax.bitcast_convert_type(
        jnp.bitwise_and(comb, jnp.int32(-65536)), jnp.float32)
    eh3 = lax.bitcast_convert_type(
        jnp.left_shift(comb, 16), jnp.float32)
    acat3b = acat3.astype(jnp.bfloat16)
    zu = jnp.dot(umlp, W1u_ref[...],
                 preferred_element_type=jnp.float32) + b1_ref[...]
    x1 = jnp.dot(acat3b.reshape(T, F), Wx_ref[...],
                 preferred_element_type=jnp.float32)
    h1 = jnp.maximum(x1.reshape(H, BB, x1.shape[1]) + zu[None, :, :], 0.0)
    h1b = h1.reshape(T, x1.shape[1]).astype(jnp.bfloat16)
    h2 = jnp.maximum(jnp.dot(h1b, W2_ref[...],
                             preferred_element_type=jnp.float32)
                     + b2_ref[...], 0.0)
    h3 = jnp.maximum(jnp.dot(h2.astype(jnp.bfloat16), W3_ref[...],
                             preferred_element_type=jnp.float32)
                     + b3_ref[...], 0.0)
    g2 = g2_ref[0, :]
    s_mlp = jnp.sum(h3.reshape(H, BB, half) * g2[None, None, :], axis=2)
    ug1 = ucat * g1x_ref[...]
    s_gmf = jnp.sum(ug1[None, :, :] * acat3, axis=2)
    scores = (s_gmf + s_mlp) * (1.0 / math.sqrt(F))
    item = item_ref[...]
    bad = (rT == item) | (rT == PAD)
    scores = jnp.where(bad, jnp.float32(-1e9), scores)
    row = lax.broadcasted_iota(jnp.int32, (H, BB), 0)
    scores = jnp.where(row >= ml_ref[0, 0], jnp.float32(-2e9), scores)
    m = jnp.max(scores, axis=0, keepdims=True)
    e = jnp.exp(scores - m)
    se = jnp.sum(e, axis=0, keepdims=True)
    q = et_ref[...] * wl_ref[...]
    vdot = jnp.sum(eh3 * q[None, :, :], axis=2)
    out_ref[...] = (jnp.sum(e * vdot, axis=0, keepdims=True) / se
                    + bl_ref[...])


def _tc_main(histT, item_row, ucat, et, comb3,
             Wx, W1u, b1, W2, b2, W3, b3, g1x, g2row, wl2d, bl2d,
             maxlen, H, PAD):
    _, B = histT.shape
    F = et.shape[1]
    D1 = Wx.shape[1]
    BB = 128
    grid = (B // BB,)
    full = lambda s: pl.BlockSpec(s, lambda i: tuple(0 for _ in s))
    return pl.pallas_call(
        functools.partial(_tc_main_body, H=H, PAD=PAD, BB=BB),
        grid=grid,
        in_specs=[
            pl.BlockSpec((H, BB), lambda i: (0, i)),
            pl.BlockSpec((1, BB), lambda i: (0, i)),
            pl.BlockSpec((BB, F), lambda i: (i, 0)),
            pl.BlockSpec((BB, F), lambda i: (i, 0)),
            pl.BlockSpec((H, BB, F), lambda i: (0, i, 0)),
            full((F, D1)),
            full(W1u.shape),
            full((1, D1)),
            full(W2.shape),
            full((1, W2.shape[1])),
            full(W3.shape),
            full((1, W3.shape[1])),
            full((1, F)),
            full((1, F // 2)),
            full((1, F)),
            full((1, 1)),
            pl.BlockSpec(memory_space=pltpu.SMEM),
        ],
        out_specs=pl.BlockSpec((1, BB), lambda i: (0, i)),
        out_shape=jax.ShapeDtypeStruct((1, B), jnp.float32),
        compiler_params=pltpu.CompilerParams(
            dimension_semantics=("arbitrary",)),
    )(histT, item_row, ucat, et, comb3,
      Wx, W1u, b1.reshape(1, -1), W2, b2.reshape(1, -1), W3,
      b3.reshape(1, -1), g1x, g2row, wl2d, bl2d, maxlen)


def kernel(user_idx, item_idx, user_hist, embed_global, embed_target,
           embed_hist, aff_user_gmf, aff_item_gmf, aff_user_mlp,
           aff_item_mlp, W1, b1, W2, b2, W3, b3, W_logit, b_logit):
    B = user_idx.shape[0]
    H = user_hist.shape[1]
    F = embed_global.shape[0]
    PAD = embed_target.shape[0] - 1
    half = F // 2

    user_idx = user_idx.astype(jnp.int32)
    item_idx = item_idx.astype(jnp.int32)
    hist_t = jnp.pad(user_hist.astype(jnp.int32), ((0, 0), (0, L - H)),
                     constant_values=PAD)
    ucat_t = jnp.concatenate([aff_user_gmf, aff_user_mlp], axis=1)
    acat_f = jnp.concatenate([aff_item_gmf, aff_item_mlp], axis=1)
    a_bits = lax.bitcast_convert_type(
        acat_f.astype(jnp.bfloat16), jnp.uint16).astype(jnp.uint32) << 16
    b_bits = lax.bitcast_convert_type(
        embed_hist.astype(jnp.bfloat16), jnp.uint16).astype(jnp.uint32)
    comb_t = lax.bitcast_convert_type(a_bits | b_bits, jnp.int32)
    Wx = jnp.concatenate([jnp.zeros_like(W1[half:, :]), W1[half:, :]],
                         axis=0).astype(jnp.bfloat16)
    W1u = W1[:half, :]
    g1x = jnp.concatenate([embed_global[:half],
                           jnp.zeros((half,), jnp.float32)]).reshape(1, -1)
    g2row = embed_global[half:].reshape(1, -1)

    uidx2 = user_idx.reshape(-1, 128)
    iidx2 = item_idx.reshape(-1, 128)

    histB = _sc_gather_hist(uidx2, hist_t)
    ucat, et = _sc_gather_user(uidx2, iidx2, ucat_t, embed_target)

    histT = histB[:, :H].T
    maxlen = _tc_maxlen(histT, PAD)
    item_row = item_idx.reshape(1, -1)

    NCHUNK = 8
    CB = B // NCHUNK
    outs = []
    for j in range(NCHUNK):
        sl = slice(j * CB, (j + 1) * CB)
        histT_c = histT[:, sl]
        comb_c = _sc_gather_items(histT_c.reshape(-1), comb_t)
        outs.append(_tc_main(
            histT_c, item_row[:, sl], ucat[sl], et[sl],
            comb_c.reshape(H, CB, F),
            Wx, W1u, b1, W2.astype(jnp.bfloat16), b2,
            W3.astype(jnp.bfloat16), b3, g1x, g2row,
            W_logit.reshape(1, -1), b_logit.reshape(1, 1),
            maxlen, H, PAD))
    return jnp.concatenate(outs, axis=1)[0]
